# Initial kernel scaffold; baseline (speedup 1.0000x reference)
import jax
import jax.numpy as jnp
from jax import lax
from jax.experimental import pallas as pl
from jax.experimental.pallas import tpu as pltpu
from jax.experimental.pallas import tpu_sc as plsc

H = 16
L = 2048
TAB = 2 * L - 1
TAB_PAD = 4096
BUF_PAD = TAB_PAD + 16
NC, NS = 2, 16
NW = NC * NS
ROWS_PER_W = (H * L) // NW
GROUPS = ROWS_PER_W // 8


def _sc_body(bias_hbm, out_hbm, buf, tshift, sem):
    cid = lax.axis_index("c")
    sid = lax.axis_index("s")
    w = cid * NS + sid
    h = w // 2
    i0 = (w % 2) * ROWS_PER_W

    pltpu.sync_copy(bias_hbm.at[h], buf.at[pl.ds(0, TAB_PAD)])

    iota16 = lax.iota(jnp.int32, 16)

    def build(k, carry):
        base = k * 16
        for b in range(8):
            idx = base + (7 - b) + iota16
            tshift[b, pl.ds(base, 16)] = plsc.load_gather(buf, [idx])
        return carry

    lax.fori_loop(0, TAB_PAD // 16, build, 0)

    def emit(g, carry):
        off = pl.multiple_of((L - 8) - i0 - 8 * g, 8)
        row0 = h * L + i0 + 8 * g
        pltpu.async_copy(
            tshift.at[:, pl.ds(off, L)], out_hbm.at[pl.ds(row0, 8), :], sem
        )

        @pl.when(g >= 2)
        def _():
            pltpu.make_async_copy(
                tshift.at[:, pl.ds(0, L)], out_hbm.at[pl.ds(0, 8), :], sem
            ).wait()

        return carry

    lax.fori_loop(0, GROUPS, emit, 0)

    for _ in range(2):
        pltpu.make_async_copy(
            tshift.at[:, pl.ds(0, L)], out_hbm.at[pl.ds(0, 8), :], sem
        ).wait()


def kernel(bias, length):
    del length
    bias_t = jnp.zeros((H, TAB_PAD), jnp.float32).at[:, :TAB].set(bias.T)

    fn = pl.kernel(
        _sc_body,
        out_type=jax.ShapeDtypeStruct((H * L, L), jnp.float32),
        mesh=plsc.VectorSubcoreMesh(core_axis_name="c", subcore_axis_name="s"),
        scratch_types=[
            pltpu.VMEM((BUF_PAD,), jnp.float32),
            pltpu.VMEM((8, BUF_PAD), jnp.float32),
            pltpu.SemaphoreType.DMA,
        ],
    )
    out = fn(bias_t)
    return out.reshape(H, L, L)


# speedup vs baseline: 43.0243x; 43.0243x over previous
import jax
import jax.numpy as jnp
from jax import lax
from jax.experimental import pallas as pl
from jax.experimental.pallas import tpu as pltpu
from jax.experimental.pallas import tpu_sc as plsc

H = 16
L = 2048
TAB = 2 * L - 1
TAB_PAD = 4096
BUF_PAD = TAB_PAD + 16
NC, NS = 2, 16
NW = NC * NS
ROWS_PER_W = (H * L) // NW
GROUPS = ROWS_PER_W // 8


def _sc_body(bias_hbm, out_hbm, buf, tshift, sem):
    cid = lax.axis_index("c")
    sid = lax.axis_index("s")
    w = cid * NS + sid
    h = w // 2
    i0 = (w % 2) * ROWS_PER_W

    pltpu.sync_copy(bias_hbm.at[h], buf.at[pl.ds(0, TAB_PAD)])

    iota16 = lax.iota(jnp.int32, 16)

    def build(k, carry):
        base = k * 16
        for b in range(8):
            idx = base + (7 - b) + iota16
            tshift[pl.ds(b * BUF_PAD + base, 16)] = plsc.load_gather(buf, [idx])
        return carry

    lax.fori_loop(0, TAB_PAD // 16, build, 0)

    def emit(g, carry):
        off = pl.multiple_of((L - 8) - i0 - 8 * g, 8)
        row0 = h * L + i0 + 8 * g
        for b in range(8):
            pltpu.async_copy(
                tshift.at[pl.ds(b * BUF_PAD + off, L)],
                out_hbm.at[pl.ds((row0 + b) * L, L)],
                sem,
            )

        @pl.when(g >= 2)
        def _():
            pltpu.make_async_copy(
                tshift.at[pl.ds(0, 8 * L)], out_hbm.at[pl.ds(0, 8 * L)], sem
            ).wait()

        return carry

    lax.fori_loop(0, GROUPS, emit, 0)

    for _ in range(2):
        pltpu.make_async_copy(
            tshift.at[pl.ds(0, 8 * L)], out_hbm.at[pl.ds(0, 8 * L)], sem
        ).wait()


def kernel(bias, length):
    del length
    bias_t = jnp.zeros((H, TAB_PAD), jnp.float32).at[:, :TAB].set(bias.T)

    fn = pl.kernel(
        _sc_body,
        out_type=jax.ShapeDtypeStruct((H * L * L,), jnp.float32),
        mesh=plsc.VectorSubcoreMesh(core_axis_name="c", subcore_axis_name="s"),
        scratch_types=[
            pltpu.VMEM((BUF_PAD,), jnp.float32),
            pltpu.VMEM((8 * BUF_PAD,), jnp.float32),
            pltpu.SemaphoreType.DMA,
        ],
        compiler_params=pltpu.CompilerParams(needs_layout_passes=False),
    )
    out = fn(bias_t)
    return out.reshape(H, L, L)
